# Initial kernel scaffold; baseline (speedup 1.0000x reference)
import jax
import jax.numpy as jnp
from jax.experimental import pallas as pl


def kernel(x_nchw, conv1_w, conv1_b, conv2_w, conv2_b, fc1_w, fc1_b, fc2_w, fc2_b):
    raise NotImplementedError("write your pallas kernel here")


# speedup vs baseline: 8.8076x; 8.8076x over previous
import jax
import jax.numpy as jnp
from jax.experimental import pallas as pl
from jax.experimental.pallas import tpu as pltpu

_NB = 4
_BM = 256


def _band_w1(conv1_w, parity):
    E = jnp.eye(64, dtype=conv1_w.dtype)[parity::2]
    blocks = []
    for kh in range(3):
        acc = jnp.zeros((256, 1024), conv1_w.dtype)
        for kw in range(3):
            blk = jnp.einsum('uv,co->vcuo', E, conv1_w[kh * 3 + kw])
            acc = acc + jnp.pad(blk.reshape(192, 1024),
                                ((3 * kw, 64 - 3 * kw), (0, 0)))
        blocks.append(acc)
    return jnp.concatenate(blocks, axis=0)


def _band_w2(conv2_w, parity):
    E = jnp.eye(32, dtype=conv2_w.dtype)[parity::2]
    blocks = []
    for kh in range(3):
        acc = jnp.zeros((1152, 1024), conv2_w.dtype)
        for kw in range(3):
            blk = jnp.einsum('uv,co->vcuo', E, conv2_w[kh * 3 + kw])
            acc = acc + jnp.pad(blk.reshape(1024, 1024),
                                ((32 * kw, 128 - 32 * kw), (0, 0)))
        blocks.append(acc)
    return jnp.concatenate(blocks, axis=0)


def _conv_stage_kernel(x_ref, w1e_ref, w1o_ref, b1_ref, w2e_ref, w2o_ref,
                       b2_ref, o_ref, s1_ref, s2_ref, s3_ref):
    NB = x_ref.shape[0]

    x = x_ref[...]
    l1 = jnp.concatenate([x[:, kh:kh + 64, :] for kh in range(3)],
                         axis=-1).reshape(NB * 64, 768)
    ce = jnp.dot(l1, w1e_ref[...], preferred_element_type=jnp.float32)
    co = jnp.dot(l1, w1o_ref[...], preferred_element_type=jnp.float32)
    wp = jnp.maximum(ce, co)

    s1_ref[...] = wp.reshape(NB, 32, 2, 1024)
    hp = jnp.maximum(s1_ref[:, :, 0, :], s1_ref[:, :, 1, :])
    z = jnp.maximum(hp + b1_ref[...], 0.0).astype(jnp.bfloat16)

    s2_ref[...] = jnp.zeros_like(s2_ref)
    s2_ref[:, pl.ds(1, 32), pl.ds(32, 1024)] = z
    xp2 = s2_ref[...]

    l2 = jnp.concatenate([xp2[:, kh:kh + 32, :] for kh in range(3)],
                         axis=-1).reshape(NB * 32, 3456)
    ce2 = jnp.dot(l2, w2e_ref[...], preferred_element_type=jnp.float32)
    co2 = jnp.dot(l2, w2o_ref[...], preferred_element_type=jnp.float32)
    wp2 = jnp.maximum(ce2, co2)

    s3_ref[...] = wp2.reshape(NB, 16, 2, 1024)
    hp2 = jnp.maximum(s3_ref[:, :, 0, :], s3_ref[:, :, 1, :])
    o_ref[...] = jnp.maximum(hp2 + b2_ref[...], 0.0).astype(jnp.bfloat16)


def _conv_stage(x256, w1e, w1o, b1t, w2e, w2o, b2t):
    B = x256.shape[0]
    nb = _NB if B % _NB == 0 else 1
    return pl.pallas_call(
        _conv_stage_kernel,
        out_shape=jax.ShapeDtypeStruct((B, 16, 1024), jnp.bfloat16),
        grid=(B // nb,),
        in_specs=[
            pl.BlockSpec((nb, 66, 256), lambda b: (b, 0, 0)),
            pl.BlockSpec((768, 1024), lambda b: (0, 0)),
            pl.BlockSpec((768, 1024), lambda b: (0, 0)),
            pl.BlockSpec((1, 1024), lambda b: (0, 0)),
            pl.BlockSpec((3456, 1024), lambda b: (0, 0)),
            pl.BlockSpec((3456, 1024), lambda b: (0, 0)),
            pl.BlockSpec((1, 1024), lambda b: (0, 0)),
        ],
        out_specs=pl.BlockSpec((nb, 16, 1024), lambda b: (b, 0, 0)),
        scratch_shapes=[
            pltpu.VMEM((nb, 32, 2, 1024), jnp.float32),
            pltpu.VMEM((nb, 34, 1152), jnp.bfloat16),
            pltpu.VMEM((nb, 16, 2, 1024), jnp.float32),
        ],
        compiler_params=pltpu.CompilerParams(
            dimension_semantics=("parallel",),
            vmem_limit_bytes=56 * 1024 * 1024,
        ),
    )(x256, w1e, w1o, b1t, w2e, w2o, b2t)


def _fc_head_kernel(x_ref, w1_ref, b1_ref, w2_ref, b2_ref, o_ref):
    h = jnp.dot(x_ref[...], w1_ref[...],
                preferred_element_type=jnp.float32)
    h = jnp.maximum(h + b1_ref[...], 0.0)
    logits = jnp.dot(h, w2_ref[...],
                     preferred_element_type=jnp.float32) + b2_ref[...]
    m = jnp.max(logits, axis=-1, keepdims=True)
    e = jnp.exp(logits - m)
    o_ref[...] = e / jnp.sum(e, axis=-1, keepdims=True)


def _fc_head(x, w1, b1, w2, b2):
    B, K = x.shape
    H = w1.shape[1]
    N = w2.shape[1]
    bm = _BM if B % _BM == 0 else B
    return pl.pallas_call(
        _fc_head_kernel,
        out_shape=jax.ShapeDtypeStruct((B, N), jnp.float32),
        grid=(B // bm,),
        in_specs=[
            pl.BlockSpec((bm, K), lambda b: (b, 0)),
            pl.BlockSpec((K, H), lambda b: (0, 0)),
            pl.BlockSpec((1, H), lambda b: (0, 0)),
            pl.BlockSpec((H, N), lambda b: (0, 0)),
            pl.BlockSpec((1, N), lambda b: (0, 0)),
        ],
        out_specs=pl.BlockSpec((bm, N), lambda b: (b, 0)),
        compiler_params=pltpu.CompilerParams(
            dimension_semantics=("parallel",),
            vmem_limit_bytes=56 * 1024 * 1024,
        ),
    )(x, w1, b1, w2, b2)


@jax.jit
def _forward(x_nchw, conv1_w, conv1_b, conv2_w, conv2_b,
             fc1_w, fc1_b, fc2_w, fc2_b):
    B = x_nchw.shape[0]
    x = jnp.transpose(x_nchw, (0, 2, 3, 1))
    xp = jnp.pad(x, ((0, 0), (1, 1), (1, 1), (0, 0))).astype(jnp.bfloat16)
    x256 = jnp.pad(xp.reshape(B, 66, 198), ((0, 0), (0, 0), (0, 58)))
    w1e = _band_w1(conv1_w, 0).astype(jnp.bfloat16)
    w1o = _band_w1(conv1_w, 1).astype(jnp.bfloat16)
    w2e = _band_w2(conv2_w, 0).astype(jnp.bfloat16)
    w2o = _band_w2(conv2_w, 1).astype(jnp.bfloat16)
    b1t = jnp.tile(conv1_b, (1, 32))
    b2t = jnp.tile(conv2_b, (1, 16))
    feats = _conv_stage(x256, w1e, w1o, b1t, w2e, w2o, b2t)
    flat = feats.reshape(B, 16 * 1024)
    return _fc_head(flat, fc1_w.astype(jnp.bfloat16), fc1_b, fc2_w, fc2_b)


def kernel(x_nchw, conv1_w, conv1_b, conv2_w, conv2_b,
           fc1_w, fc1_b, fc2_w, fc2_b):
    return _forward(x_nchw, conv1_w, conv1_b, conv2_w, conv2_b,
                    fc1_w, fc1_b, fc2_w, fc2_b)
